# baseline (speedup 1.0000x reference)
import functools

import jax
import jax.numpy as jnp
from jax import lax
from jax.experimental import pallas as pl
from jax.experimental.pallas import tpu as pltpu
from jax.experimental.pallas import tpu_sc as plsc

_B = 4
_SEQ = 2048
_D = 768
_EPS = 1e-12
_NC = 2
_NS = 16
_NW = _NC * _NS
_SW = _SEQ // _NW
_CH = 32
_NBUF = 4

_mesh = plsc.VectorSubcoreMesh(core_axis_name="c", subcore_axis_name="s")


def _make_sc_gather(nb):
    nchk = (nb * _SW) // _CH

    @functools.partial(
        pl.kernel,
        mesh=_mesh,
        out_type=jax.ShapeDtypeStruct((nb * _SEQ, _D), jnp.float32),
        scratch_types=[
            pltpu.VMEM((_NBUF, _CH), jnp.int32),
            pltpu.VMEM((_NBUF * _CH, _D), jnp.float32),
            pltpu.SemaphoreType.DMA((_NBUF,)),
            pltpu.SemaphoreType.DMA((_NBUF,)),
        ],
    )
    def sc_gather(ids_hbm, word_hbm, out_hbm, idx_v, ring, gsem, wsem):
        wid = lax.axis_index("s") * _NC + lax.axis_index("c")
        s0 = wid * _SW

        def tok_base(c):
            return (c % nb) * _SEQ + s0 + (c // nb) * _CH

        def buf(u):
            return ring.at[pl.ds(u * _CH, _CH)]

        def arm_gather(c, u):
            pltpu.sync_copy(ids_hbm.at[pl.ds(tok_base(c), _CH)], idx_v.at[u])
            pltpu.make_async_copy(
                word_hbm.at[idx_v.at[u]], buf(u), gsem.at[u]).start()

        def g_wait(u):
            pltpu.make_async_copy(
                word_hbm.at[idx_v.at[u]], buf(u), gsem.at[u]).wait()

        def w_desc(c, u):
            return pltpu.make_async_copy(
                buf(u), out_hbm.at[pl.ds(tok_base(c), _CH)], wsem.at[u])

        arm_gather(0, 0)
        arm_gather(1, 1)

        def pipe(c, carry):
            u = c % _NBUF
            @pl.when(c + 2 < nchk)
            def _():
                u2 = (c + 2) % _NBUF

                @pl.when(c >= 2)
                def _():
                    w_desc(c - 2, u2).wait()

                arm_gather(c + 2, u2)

            g_wait(u)
            w_desc(c, u).start()
            return carry

        lax.fori_loop(0, nchk, pipe, 0)
        for u in range(min(_NBUF, nchk)):
            w_desc(nchk - min(_NBUF, nchk) + u, u).wait()

    return sc_gather


_ROWS_BLK = 1024


def _tc_ln_body(x_ref, pos_ref, g_ref, b_ref, out_ref):
    y = x_ref[...] + pos_ref[...]
    mu = jnp.mean(y, axis=1, keepdims=True)
    d = y - mu
    var = jnp.mean(d * d, axis=1, keepdims=True)
    o = d * lax.rsqrt(var + _EPS)
    out_ref[...] = o * g_ref[...] + b_ref[...]


def _make_tc_ln(nb):
    return pl.pallas_call(
        _tc_ln_body,
        grid=(nb * _SEQ // _ROWS_BLK,),
        in_specs=[
            pl.BlockSpec((_ROWS_BLK, _D), lambda i: (i, 0)),
            pl.BlockSpec((_ROWS_BLK, _D),
                         lambda i: (i % (_SEQ // _ROWS_BLK), 0)),
            pl.BlockSpec((1, _D), lambda i: (0, 0)),
            pl.BlockSpec((1, _D), lambda i: (0, 0)),
        ],
        out_specs=pl.BlockSpec((_ROWS_BLK, _D), lambda i: (i, 0)),
        out_shape=jax.ShapeDtypeStruct((nb * _SEQ, _D), jnp.float32),
    )


_sc_gather_half = _make_sc_gather(_B // 2)
_tc_ln_half = _make_tc_ln(_B // 2)


def kernel(input_ids, word_emb, pos_emb, gamma, beta):
    ids = input_ids.reshape(-1).astype(jnp.int32)
    half = (_B // 2) * _SEQ
    g2 = gamma.reshape(1, _D)
    b2 = beta.reshape(1, _D)
    rows0 = _sc_gather_half(ids[:half], word_emb)
    rows1 = _sc_gather_half(ids[half:], word_emb)
    out0 = _tc_ln_half(rows0, pos_emb, g2, b2)
    out1 = _tc_ln_half(rows1, pos_emb, g2, b2)
    return jnp.concatenate([out0, out1], axis=0).reshape(_B, _SEQ, _D)


# speedup vs baseline: 1.4014x; 1.4014x over previous
import functools

import jax
import jax.numpy as jnp
from jax import lax
from jax.experimental import pallas as pl
from jax.experimental.pallas import tpu as pltpu
from jax.experimental.pallas import tpu_sc as plsc

_B = 4
_SEQ = 2048
_D = 768
_EPS = 1e-12
_NC = 2
_NS = 16
_NW = _NC * _NS
_SW = _SEQ // _NW
_CH = 32
_NCHK = (_B * _SW) // _CH
_NBUF = 4

_mesh = plsc.VectorSubcoreMesh(core_axis_name="c", subcore_axis_name="s")


@functools.partial(
    pl.kernel,
    mesh=_mesh,
    out_type=jax.ShapeDtypeStruct((_B * _SEQ, _D), jnp.float32),
    scratch_types=[
        pltpu.VMEM((_NBUF, _CH), jnp.int32),
        pltpu.VMEM((_NBUF * _CH, _D), jnp.float32),
        pltpu.SemaphoreType.DMA((_NBUF,)),
        pltpu.SemaphoreType.DMA((_NBUF,)),
    ],
)
def _sc_gather(ids_hbm, word_hbm, out_hbm, idx_v, ring, gsem, wsem):
    wid = lax.axis_index("s") * _NC + lax.axis_index("c")
    s0 = wid * _SW

    def tok_base(c):
        return (c % 4) * _SEQ + s0 + (c // 4) * _CH

    def buf(u):
        return ring.at[pl.ds(u * _CH, _CH)]

    def arm_gather(c, u):
        pltpu.sync_copy(ids_hbm.at[pl.ds(tok_base(c), _CH)], idx_v.at[u])
        pltpu.make_async_copy(
            word_hbm.at[idx_v.at[u]], buf(u), gsem.at[u]).start()

    def g_wait(u):
        pltpu.make_async_copy(
            word_hbm.at[idx_v.at[u]], buf(u), gsem.at[u]).wait()

    def w_desc(c, u):
        return pltpu.make_async_copy(
            buf(u), out_hbm.at[pl.ds(tok_base(c), _CH)], wsem.at[u])

    arm_gather(0, 0)
    arm_gather(1, 1)

    def pipe(c, carry):
        u = c % _NBUF
        @pl.when(c + 2 < _NCHK)
        def _():
            u2 = (c + 2) % _NBUF

            @pl.when(c >= 2)
            def _():
                w_desc(c - 2, u2).wait()

            arm_gather(c + 2, u2)

        g_wait(u)
        w_desc(c, u).start()
        return carry

    lax.fori_loop(0, _NCHK, pipe, 0)
    for u in range(_NBUF):
        w_desc(_NCHK - _NBUF + u, u).wait()


_ROWS_BLK = 2048


def _tc_ln_body(x_ref, pos_ref, g_ref, b_ref, out_ref):
    y = x_ref[...] + pos_ref[...]
    mu = jnp.mean(y, axis=1, keepdims=True)
    d = y - mu
    var = jnp.mean(d * d, axis=1, keepdims=True)
    o = d * lax.rsqrt(var + _EPS)
    out_ref[...] = o * g_ref[...] + b_ref[...]


_tc_ln = pl.pallas_call(
    _tc_ln_body,
    grid=(_B * _SEQ // _ROWS_BLK,),
    in_specs=[
        pl.BlockSpec((_ROWS_BLK, _D), lambda i: (i, 0)),
        pl.BlockSpec((_ROWS_BLK, _D), lambda i: (i % (_SEQ // _ROWS_BLK), 0)),
        pl.BlockSpec((1, _D), lambda i: (0, 0)),
        pl.BlockSpec((1, _D), lambda i: (0, 0)),
    ],
    out_specs=pl.BlockSpec((_ROWS_BLK, _D), lambda i: (i, 0)),
    out_shape=jax.ShapeDtypeStruct((_B * _SEQ, _D), jnp.float32),
)


def kernel(input_ids, word_emb, pos_emb, gamma, beta):
    ids = input_ids.reshape(-1).astype(jnp.int32)
    rows = _sc_gather(ids, word_emb)
    out = _tc_ln(rows, pos_emb, gamma.reshape(1, _D), beta.reshape(1, _D))
    return out.reshape(_B, _SEQ, _D)
